# Initial kernel scaffold; baseline (speedup 1.0000x reference)
import jax
import jax.numpy as jnp
from jax.experimental import pallas as pl


def kernel(indices, E):
    raise NotImplementedError("write your pallas kernel here")


# speedup vs baseline: 1.5478x; 1.5478x over previous
import functools

import jax
import jax.numpy as jnp
from jax import lax
from jax.experimental import pallas as pl
from jax.experimental.pallas import tpu as pltpu
from jax.experimental.pallas import tpu_sc as plsc

BATCH = 16384
FIELDS = 26
NUM_NODES = 32

B = BATCH * FIELDS
NW = 32
B_PER_W = B // NW
IDXW = 128
CHUNK = 1024
K = CHUNK // IDXW
NCHUNK = B_PER_W // CHUNK

_mesh = plsc.VectorSubcoreMesh(core_axis_name="c", subcore_axis_name="s")


@functools.partial(
    pl.kernel,
    mesh=_mesh,
    compiler_params=pltpu.CompilerParams(use_tc_tiling_on_sc=False),
    out_type=jax.ShapeDtypeStruct((B, NUM_NODES), jnp.float32),
    scratch_types=[
        pltpu.VMEM((K, IDXW), jnp.int32),
        pltpu.VMEM((CHUNK, NUM_NODES), jnp.float32),
        pltpu.SemaphoreType.DMA,
    ],
)
def _gather_rows(idx_hbm, table_hbm, out_hbm, idx_v, rows_v, sem):
    wid = lax.axis_index("s") * 2 + lax.axis_index("c")
    base = wid * B_PER_W

    def body(c, carry):
        off = pl.multiple_of(base + c * CHUNK, CHUNK)
        pltpu.sync_copy(idx_hbm.at[pl.ds(pl.multiple_of(off // IDXW, 8), K)], idx_v)
        copies = [
            pltpu.async_copy(
                table_hbm.at[idx_v.at[j]],
                rows_v.at[pl.ds(j * IDXW, IDXW)],
                sem,
            )
            for j in range(K)
        ]
        for cp in copies:
            cp.wait()
        pltpu.sync_copy(rows_v, out_hbm.at[pl.ds(off, CHUNK)])
        return carry

    lax.fori_loop(0, NCHUNK, body, 0, unroll=False)


def kernel(indices, E):
    flat_idx = indices.reshape(B // IDXW, IDXW).astype(jnp.int32)
    out = _gather_rows(flat_idx, E)
    return out.reshape(BATCH, FIELDS, NUM_NODES)


# speedup vs baseline: 1.5521x; 1.0028x over previous
import functools

import jax
import jax.numpy as jnp
from jax import lax
from jax.experimental import pallas as pl
from jax.experimental.pallas import tpu as pltpu
from jax.experimental.pallas import tpu_sc as plsc

BATCH = 16384
FIELDS = 26
NUM_NODES = 32

B = BATCH * FIELDS
NW = 32
B_PER_W = B // NW
IDXW = 128
CHUNK = 512
K = CHUNK // IDXW
NCHUNK = B_PER_W // CHUNK
NBUF = 2

_mesh = plsc.VectorSubcoreMesh(core_axis_name="c", subcore_axis_name="s")


@functools.partial(
    pl.kernel,
    mesh=_mesh,
    compiler_params=pltpu.CompilerParams(use_tc_tiling_on_sc=False),
    out_type=jax.ShapeDtypeStruct((B, NUM_NODES), jnp.float32),
    scratch_types=[
        pltpu.VMEM((NBUF, CHUNK), jnp.int32),
        pltpu.VMEM((NBUF, CHUNK, NUM_NODES), jnp.float32),
        pltpu.SemaphoreType.DMA,
        pltpu.SemaphoreType.DMA,
        pltpu.SemaphoreType.DMA,
        pltpu.SemaphoreType.DMA,
        pltpu.SemaphoreType.DMA,
    ],
)
def _gather_rows(idx_hbm, table_hbm, out_hbm, idx_v, rows_v, sem_l0, sem_l1,
                 sem_s0, sem_s1, sem_g):
    wid = lax.axis_index("s") * 2 + lax.axis_index("c")
    base = wid * B_PER_W
    sem_l = (sem_l0, sem_l1)
    sem_s = (sem_s0, sem_s1)

    def idx_slice(c):
        start = pl.multiple_of(base + c * CHUNK, CHUNK)
        return idx_hbm.at[pl.ds(start, CHUNK)]

    def out_slice(c):
        return out_hbm.at[pl.ds(pl.multiple_of(base + c * CHUNK, CHUNK), CHUNK)]

    for b in range(NBUF):
        pltpu.async_copy(idx_slice(b), idx_v.at[b], sem_l[b])

    def body(i, carry):
        c0 = i * NBUF
        for b in range(NBUF):
            c = c0 + b
            pltpu.make_async_copy(idx_slice(c), idx_v.at[b], sem_l[b]).wait()
            @pl.when(c0 >= NBUF)
            def _():
                pltpu.make_async_copy(
                    rows_v.at[b], out_slice(c - NBUF), sem_s[b]).wait()
            copies = [
                pltpu.async_copy(
                    table_hbm.at[idx_v.at[b].at[pl.ds(j * IDXW, IDXW)]],
                    rows_v.at[b].at[pl.ds(j * IDXW, IDXW)],
                    sem_g,
                )
                for j in range(K)
            ]
            for cp in copies:
                cp.wait()
            pltpu.async_copy(rows_v.at[b], out_slice(c), sem_s[b])
            @pl.when(c + NBUF < NCHUNK)
            def _():
                pltpu.async_copy(idx_slice(c + NBUF), idx_v.at[b], sem_l[b])
        return carry

    lax.fori_loop(0, NCHUNK // NBUF, body, 0, unroll=False)

    for b in range(NBUF):
        c = NCHUNK - NBUF + b
        pltpu.make_async_copy(rows_v.at[b], out_slice(c), sem_s[b]).wait()


def kernel(indices, E):
    flat_idx = indices.reshape(B).astype(jnp.int32)
    out = _gather_rows(flat_idx, E)
    return out.reshape(BATCH, FIELDS, NUM_NODES)


# speedup vs baseline: 1.5525x; 1.0003x over previous
import functools

import jax
import jax.numpy as jnp
from jax import lax
from jax.experimental import pallas as pl
from jax.experimental.pallas import tpu as pltpu
from jax.experimental.pallas import tpu_sc as plsc

BATCH = 16384
FIELDS = 26
NUM_NODES = 32

B = BATCH * FIELDS
NW = 32
B_PER_W = B // NW
IDXW = 128
CHUNK = 512
K = CHUNK // IDXW
NCHUNK = B_PER_W // CHUNK
NBUF = 2

_mesh = plsc.VectorSubcoreMesh(core_axis_name="c", subcore_axis_name="s")


@functools.partial(
    pl.kernel,
    mesh=_mesh,
    compiler_params=pltpu.CompilerParams(use_tc_tiling_on_sc=False),
    out_type=jax.ShapeDtypeStruct((B, NUM_NODES), jnp.float32),
    scratch_types=[
        pltpu.VMEM((NBUF, CHUNK), jnp.int32),
        pltpu.VMEM((NBUF, CHUNK, NUM_NODES), jnp.float32),
        pltpu.SemaphoreType.DMA,
        pltpu.SemaphoreType.DMA,
        pltpu.SemaphoreType.DMA,
        pltpu.SemaphoreType.DMA,
        pltpu.SemaphoreType.DMA,
    ],
)
def _gather_rows(idx_hbm, table_hbm, out_hbm, idx_v, rows_v, sem_l0, sem_l1,
                 sem_s0, sem_s1, sem_g):
    wid = lax.axis_index("s") * 2 + lax.axis_index("c")
    base = wid * B_PER_W
    sem_l = (sem_l0, sem_l1)
    sem_s = (sem_s0, sem_s1)

    def idx_slice(c):
        start = pl.multiple_of(base + c * CHUNK, CHUNK)
        return idx_hbm.at[pl.ds(start, CHUNK)]

    def out_slice(c):
        return out_hbm.at[pl.ds(pl.multiple_of(base + c * CHUNK, CHUNK), CHUNK)]

    for b in range(NBUF):
        pltpu.async_copy(idx_slice(b), idx_v.at[b], sem_l[b])

    def body(i, carry):
        c0 = i * NBUF
        for b in range(NBUF):
            c = c0 + b
            pltpu.make_async_copy(idx_slice(c), idx_v.at[b], sem_l[b]).wait()
            @pl.when(c0 >= NBUF)
            def _():
                pltpu.make_async_copy(
                    rows_v.at[b], out_slice(c - NBUF), sem_s[b]).wait()
            pltpu.async_copy(
                table_hbm.at[idx_v.at[b]], rows_v.at[b], sem_g,
            ).wait()
            pltpu.async_copy(rows_v.at[b], out_slice(c), sem_s[b])
            @pl.when(c + NBUF < NCHUNK)
            def _():
                pltpu.async_copy(idx_slice(c + NBUF), idx_v.at[b], sem_l[b])
        return carry

    lax.fori_loop(0, NCHUNK // NBUF, body, 0, unroll=False)

    for b in range(NBUF):
        c = NCHUNK - NBUF + b
        pltpu.make_async_copy(rows_v.at[b], out_slice(c), sem_s[b]).wait()


def kernel(indices, E):
    flat_idx = indices.reshape(B).astype(jnp.int32)
    out = _gather_rows(flat_idx, E)
    return out.reshape(BATCH, FIELDS, NUM_NODES)
